# baseline (speedup 1.0000x reference)
import functools

import jax
import jax.numpy as jnp
from jax import lax
from jax.experimental import pallas as pl
from jax.experimental.pallas import tpu as pltpu
from jax.experimental.pallas import tpu_sc as plsc

NE = 8192
DIM = 256
BETA = 0.25
BM = 512
JW = 2736


def _dist_argmin_body(z_ref, e_ref, z2_ref, e2_ref, idx_ref, dsum_ref):
    m = pl.program_id(0)
    mm = lax.dot_general(z_ref[...], e_ref[...], (((1,), (1,)), ((), ())),
                         preferred_element_type=jnp.float32)
    dist = (z2_ref[...] - 2.0 * mm) + e2_ref[...]
    cols = lax.broadcasted_iota(jnp.int32, dist.shape, 1)

    inf = jnp.float32(jnp.inf)
    run_v = jnp.full((BM, 1), inf, jnp.float32)
    best_f = jnp.full((BM, 1), inf, jnp.float32)
    run_i = jnp.zeros((BM, 1), jnp.int32)
    for w0 in range(0, NE, JW):
        w1 = min(w0 + JW, NE)
        mask = (cols >= w0) & (cols < w1)
        dw = jnp.where(mask, dist, inf)
        mv = jnp.min(dw, axis=1, keepdims=True)
        mi = jnp.min(jnp.where(dw == mv, cols, NE), axis=1, keepdims=True)
        take = mv < run_v
        run_i = jnp.where(take, mi, run_i)
        best_f = jnp.where(take, mv, best_f)
        run_v = jnp.where(take, mv, run_v).astype(jnp.bfloat16).astype(jnp.float32)
    idx_ref[0, 0, :] = run_i[:, 0]

    @pl.when(m == 0)
    def _():
        dsum_ref[0, 0] = 0.0

    dsum_ref[0, 0] += jnp.sum(best_f)


def _dist_argmin(zb, eb, z2, e2):
    n_blocks = zb.shape[0] // BM
    idx_blocks, dsum = pl.pallas_call(
        _dist_argmin_body,
        grid=(n_blocks,),
        in_specs=[
            pl.BlockSpec((BM, DIM), lambda m: (m, 0)),
            pl.BlockSpec((NE, DIM), lambda m: (0, 0)),
            pl.BlockSpec((BM, 1), lambda m: (m, 0)),
            pl.BlockSpec((1, NE), lambda m: (0, 0)),
        ],
        out_specs=[
            pl.BlockSpec((1, 1, BM), lambda m: (m, 0, 0)),
            pl.BlockSpec(memory_space=pltpu.SMEM),
        ],
        out_shape=[
            jax.ShapeDtypeStruct((n_blocks, 1, BM), jnp.int32),
            jax.ShapeDtypeStruct((1, 1), jnp.float32),
        ],
    )(zb, eb, z2, e2)
    return idx_blocks.reshape(-1), dsum[0, 0]


def _sc_gather(embedding, indices):
    info = plsc.get_sparse_core_info()
    nw = info.num_cores * info.num_subcores
    n = indices.shape[0]
    bpw = n // nw
    ch = 128
    mesh = plsc.VectorSubcoreMesh(core_axis_name="c", subcore_axis_name="s")

    @functools.partial(
        pl.kernel,
        mesh=mesh,
        out_type=jax.ShapeDtypeStruct((n, DIM), jnp.float32),
        scratch_types=[
            pltpu.VMEM((bpw,), jnp.int32),
            pltpu.VMEM((2, ch, DIM), jnp.float32),
            pltpu.SemaphoreType.DMA,
            pltpu.SemaphoreType.DMA,
        ],
    )
    def k(e_hbm, idx_hbm, out_hbm, idx_v, rows_v, sem0, sem1):
        wid = lax.axis_index("s") * info.num_cores + lax.axis_index("c")
        base = wid * bpw
        pltpu.sync_copy(idx_hbm.at[pl.ds(base, bpw)], idx_v)
        sems = (sem0, sem1)
        nch = bpw // ch
        cps = [pltpu.async_copy(e_hbm.at[idx_v.at[pl.ds(0, ch)]],
                                rows_v.at[0], sems[0]), None]
        for c in range(nch):
            nxt = c + 1
            if nxt < nch:
                cps[nxt % 2] = pltpu.async_copy(
                    e_hbm.at[idx_v.at[pl.ds(nxt * ch, ch)]],
                    rows_v.at[nxt % 2], sems[nxt % 2])
            cps[c % 2].wait()
            pltpu.sync_copy(rows_v.at[c % 2],
                            out_hbm.at[pl.ds(base + c * ch, ch)])

    return k(embedding, indices)


def kernel(z, embedding):
    B, D, H, W = z.shape
    zt = jnp.transpose(z, (0, 2, 3, 1))
    z_flat = zt.reshape(-1, D)
    z2 = jnp.sum(zt ** 2, axis=3).reshape(-1, 1)
    e2 = jnp.sum(embedding ** 2, axis=1).reshape(1, -1)
    zb = z_flat.astype(jnp.bfloat16)
    eb = embedding.astype(jnp.bfloat16)
    indices, dsum = _dist_argmin(zb, eb, z2, e2)
    loss = (1.0 + BETA) * dsum / z.size
    q_flat = _sc_gather(embedding, indices)
    quantized = jnp.transpose(q_flat.reshape(B, H, W, D), (0, 3, 1, 2))
    return quantized, loss, indices


# speedup vs baseline: 1.3588x; 1.3588x over previous
import functools

import jax
import jax.numpy as jnp
from jax import lax
from jax.experimental import pallas as pl
from jax.experimental.pallas import tpu as pltpu
from jax.experimental.pallas import tpu_sc as plsc

NE = 8192
DIM = 256
BETA = 0.25
BM = 512
JW = 2736


def _dist_argmin_body(z_ref, e_ref, z2_ref, e2_ref, idx_ref, dsum_ref):
    m = pl.program_id(0)
    mm = lax.dot_general(z_ref[...], e_ref[...], (((1,), (1,)), ((), ())),
                         preferred_element_type=jnp.float32)
    dist = (z2_ref[...] + mm) + e2_ref[...]
    cols = lax.broadcasted_iota(jnp.int32, dist.shape, 1)

    inf = jnp.float32(jnp.inf)
    c1 = cols < JW
    c2 = cols < 2 * JW
    mv0 = jnp.min(jnp.where(c1, dist, inf), axis=1, keepdims=True)
    mv1 = jnp.min(jnp.where(c1, inf, jnp.where(c2, dist, inf)),
                  axis=1, keepdims=True)
    mv2 = jnp.min(jnp.where(c2, inf, dist), axis=1, keepdims=True)
    r0 = mv0.astype(jnp.bfloat16).astype(jnp.float32)
    t1 = mv1 < r0
    r1 = jnp.where(t1, mv1, r0).astype(jnp.bfloat16).astype(jnp.float32)
    t2 = mv2 < r1
    best_f = jnp.where(t2, mv2, jnp.where(t1, mv1, mv0))
    win = jnp.where(t2, 2, jnp.where(t1, 1, 0)).astype(jnp.int32)
    wcol = jnp.where(c1, 0, jnp.where(c2, 1, 2)).astype(jnp.int32)
    cand = (dist == best_f) & (wcol == win)
    idx = jnp.min(jnp.where(cand, cols, NE), axis=1)
    idx_ref[0, 0, :] = idx

    @pl.when(m == 0)
    def _():
        dsum_ref[0, 0] = 0.0

    dsum_ref[0, 0] += jnp.sum(best_f)


def _dist_argmin(zb, eb, z2, e2):
    n_blocks = zb.shape[0] // BM
    idx_blocks, dsum = pl.pallas_call(
        _dist_argmin_body,
        grid=(n_blocks,),
        in_specs=[
            pl.BlockSpec((BM, DIM), lambda m: (m, 0)),
            pl.BlockSpec((NE, DIM), lambda m: (0, 0)),
            pl.BlockSpec((BM, 1), lambda m: (m, 0)),
            pl.BlockSpec((1, NE), lambda m: (0, 0)),
        ],
        out_specs=[
            pl.BlockSpec((1, 1, BM), lambda m: (m, 0, 0)),
            pl.BlockSpec(memory_space=pltpu.SMEM),
        ],
        out_shape=[
            jax.ShapeDtypeStruct((n_blocks, 1, BM), jnp.int32),
            jax.ShapeDtypeStruct((1, 1), jnp.float32),
        ],
    )(zb, eb, z2, e2)
    return idx_blocks.reshape(-1), dsum[0, 0]


def _sc_gather(embedding, indices):
    info = plsc.get_sparse_core_info()
    nw = info.num_cores * info.num_subcores
    n = indices.shape[0]
    bpw = n // nw
    ch = 128
    mesh = plsc.VectorSubcoreMesh(core_axis_name="c", subcore_axis_name="s")

    @functools.partial(
        pl.kernel,
        mesh=mesh,
        out_type=jax.ShapeDtypeStruct((n, DIM), jnp.float32),
        scratch_types=[
            pltpu.VMEM((bpw,), jnp.int32),
            pltpu.VMEM((2, ch, DIM), jnp.float32),
            pltpu.SemaphoreType.DMA,
            pltpu.SemaphoreType.DMA,
        ],
    )
    def k(e_hbm, idx_hbm, out_hbm, idx_v, rows_v, sem0, sem1):
        wid = lax.axis_index("s") * info.num_cores + lax.axis_index("c")
        base = wid * bpw
        pltpu.sync_copy(idx_hbm.at[pl.ds(base, bpw)], idx_v)
        sems = (sem0, sem1)
        nch = bpw // ch
        cps = [pltpu.async_copy(e_hbm.at[idx_v.at[pl.ds(0, ch)]],
                                rows_v.at[0], sems[0]), None]
        for c in range(nch):
            nxt = c + 1
            if nxt < nch:
                cps[nxt % 2] = pltpu.async_copy(
                    e_hbm.at[idx_v.at[pl.ds(nxt * ch, ch)]],
                    rows_v.at[nxt % 2], sems[nxt % 2])
            cps[c % 2].wait()
            pltpu.sync_copy(rows_v.at[c % 2],
                            out_hbm.at[pl.ds(base + c * ch, ch)])

    return k(embedding, indices)


def kernel(z, embedding):
    B, D, H, W = z.shape
    zt = jnp.transpose(z, (0, 2, 3, 1))
    z_flat = zt.reshape(-1, D)
    z2 = jnp.sum(zt ** 2, axis=3).reshape(-1, 1)
    e2 = jnp.sum(embedding ** 2, axis=1).reshape(1, -1)
    zb = (z_flat * jnp.float32(-2.0)).astype(jnp.bfloat16)
    eb = embedding.astype(jnp.bfloat16)
    indices, dsum = _dist_argmin(zb, eb, z2, e2)
    loss = (1.0 + BETA) * dsum / z.size
    q_flat = _sc_gather(embedding, indices)
    quantized = jnp.transpose(q_flat.reshape(B, H, W, D), (0, 3, 1, 2))
    return quantized, loss, indices


# speedup vs baseline: 1.3784x; 1.0144x over previous
import functools

import jax
import jax.numpy as jnp
from jax import lax
from jax.experimental import pallas as pl
from jax.experimental.pallas import tpu as pltpu
from jax.experimental.pallas import tpu_sc as plsc

NE = 8192
DIM = 256
BETA = 0.25
BM = 1024
JW = 2736


def _dist_argmin_body(z_ref, e_ref, z2_ref, e2_ref, idx_ref, dsum_ref):
    m = pl.program_id(0)
    mm = lax.dot_general(z_ref[...], e_ref[...], (((1,), (1,)), ((), ())),
                         preferred_element_type=jnp.float32)
    dist = (z2_ref[...] + mm) + e2_ref[...]
    cols = lax.broadcasted_iota(jnp.int32, dist.shape, 1)

    inf = jnp.float32(jnp.inf)
    c1 = cols < JW
    c2 = cols < 2 * JW
    mv0 = jnp.min(jnp.where(c1, dist, inf), axis=1, keepdims=True)
    mv1 = jnp.min(jnp.where(c1, inf, jnp.where(c2, dist, inf)),
                  axis=1, keepdims=True)
    mv2 = jnp.min(jnp.where(c2, inf, dist), axis=1, keepdims=True)
    r0 = mv0.astype(jnp.bfloat16).astype(jnp.float32)
    t1 = mv1 < r0
    r1 = jnp.where(t1, mv1, r0).astype(jnp.bfloat16).astype(jnp.float32)
    t2 = mv2 < r1
    best_f = jnp.where(t2, mv2, jnp.where(t1, mv1, mv0))
    win = jnp.where(t2, 2, jnp.where(t1, 1, 0)).astype(jnp.int32)
    wcol = jnp.where(c1, 0, jnp.where(c2, 1, 2)).astype(jnp.int32)
    cand = (dist == best_f) & (wcol == win)
    idx = jnp.min(jnp.where(cand, cols, NE), axis=1)
    idx_ref[0, 0, :] = idx

    @pl.when(m == 0)
    def _():
        dsum_ref[0, 0] = 0.0

    dsum_ref[0, 0] += jnp.sum(best_f)


def _dist_argmin(zb, eb, z2, e2):
    n_blocks = zb.shape[0] // BM
    idx_blocks, dsum = pl.pallas_call(
        _dist_argmin_body,
        grid=(n_blocks,),
        in_specs=[
            pl.BlockSpec((BM, DIM), lambda m: (m, 0)),
            pl.BlockSpec((NE, DIM), lambda m: (0, 0)),
            pl.BlockSpec((BM, 1), lambda m: (m, 0)),
            pl.BlockSpec((1, NE), lambda m: (0, 0)),
        ],
        out_specs=[
            pl.BlockSpec((1, 1, BM), lambda m: (m, 0, 0)),
            pl.BlockSpec(memory_space=pltpu.SMEM),
        ],
        out_shape=[
            jax.ShapeDtypeStruct((n_blocks, 1, BM), jnp.int32),
            jax.ShapeDtypeStruct((1, 1), jnp.float32),
        ],
    )(zb, eb, z2, e2)
    return idx_blocks.reshape(-1), dsum[0, 0]


def _sc_gather(embedding, indices):
    info = plsc.get_sparse_core_info()
    nw = info.num_cores * info.num_subcores
    n = indices.shape[0]
    bpw = n // nw
    ch = 128
    mesh = plsc.VectorSubcoreMesh(core_axis_name="c", subcore_axis_name="s")

    @functools.partial(
        pl.kernel,
        mesh=mesh,
        out_type=jax.ShapeDtypeStruct((n, DIM), jnp.float32),
        scratch_types=[
            pltpu.VMEM((bpw,), jnp.int32),
            pltpu.VMEM((2, ch, DIM), jnp.float32),
            pltpu.SemaphoreType.DMA,
            pltpu.SemaphoreType.DMA,
        ],
    )
    def k(e_hbm, idx_hbm, out_hbm, idx_v, rows_v, sem0, sem1):
        wid = lax.axis_index("s") * info.num_cores + lax.axis_index("c")
        base = wid * bpw
        pltpu.sync_copy(idx_hbm.at[pl.ds(base, bpw)], idx_v)
        sems = (sem0, sem1)
        nch = bpw // ch
        cps = [pltpu.async_copy(e_hbm.at[idx_v.at[pl.ds(0, ch)]],
                                rows_v.at[0], sems[0]), None]
        for c in range(nch):
            nxt = c + 1
            if nxt < nch:
                cps[nxt % 2] = pltpu.async_copy(
                    e_hbm.at[idx_v.at[pl.ds(nxt * ch, ch)]],
                    rows_v.at[nxt % 2], sems[nxt % 2])
            cps[c % 2].wait()
            pltpu.sync_copy(rows_v.at[c % 2],
                            out_hbm.at[pl.ds(base + c * ch, ch)])

    return k(embedding, indices)


def kernel(z, embedding):
    B, D, H, W = z.shape
    zt = jnp.transpose(z, (0, 2, 3, 1))
    z_flat = zt.reshape(-1, D)
    z2 = jnp.sum(zt ** 2, axis=3).reshape(-1, 1)
    e2 = jnp.sum(embedding ** 2, axis=1).reshape(1, -1)
    zb = (z_flat * jnp.float32(-2.0)).astype(jnp.bfloat16)
    eb = embedding.astype(jnp.bfloat16)
    indices, dsum = _dist_argmin(zb, eb, z2, e2)
    loss = (1.0 + BETA) * dsum / z.size
    q_flat = _sc_gather(embedding, indices)
    quantized = jnp.transpose(q_flat.reshape(B, H, W, D), (0, 3, 1, 2))
    return quantized, loss, indices
